# baseline (speedup 1.0000x reference)
import jax
import jax.numpy as jnp
from jax import lax
from jax.experimental import pallas as pl
from jax.experimental.pallas import tpu as pltpu

_B, _T, _R, _C = 8, 50, 5000, 81
_RED = 16.0
_N_POS = 128
_N_NEG = 384
_RT = 1000
_NR = _R // _RT

_PCNT, _NCNT, _CLS, _ACC, _REG, _CE0P, _CE0N, _A0P, _A0N, _R0 = range(10)
_NSLOT = 10


def _body(nms_ref, reg_ref, cls_ref, bb_ref, cl_ref,
          out_cls, out_reg, out_acc, s_ref, tri_ref):
    b = pl.program_id(0)
    i = pl.program_id(1)
    first = jnp.logical_and(b == 0, i == 0)
    last = jnp.logical_and(b == _B - 1, i == _NR - 1)

    @pl.when(first)
    def _():
        for k in range(_NSLOT):
            s_ref[k] = 0.0
        ri = lax.broadcasted_iota(jnp.int32, (_RT, _RT), 0)
        ci = lax.broadcasted_iota(jnp.int32, (_RT, _RT), 1)
        tri_ref[...] = (ri > ci).astype(jnp.float32)

    pcnt = s_ref[_PCNT]
    ncnt = s_ref[_NCNT]
    active = jnp.logical_or(pcnt < float(_N_POS), ncnt < float(_N_NEG))

    @pl.when(active)
    def _():
        nms = nms_ref[0]
        top, left = nms[:, 0:1], nms[:, 1:2]
        bot, right = nms[:, 2:3], nms[:, 3:4]
        bb = bb_ref[0]
        bt, bl = bb[0:1, :], bb[1:2, :]
        bbm, br = bb[2:3, :], bb[3:4, :]
        it = jnp.maximum(top, bt)
        il = jnp.maximum(left, bl)
        ib = jnp.minimum(bot, bbm)
        ir = jnp.minimum(right, br)
        inter = jnp.maximum(ib - it, 0.0) * jnp.maximum(ir - il, 0.0)
        area_a = (bot - top) * (right - left)
        area_b = (bbm - bt) * (br - bl)
        iou = inter / (area_a + area_b - inter + 1e-8)
        iou_max = jnp.max(iou, axis=1, keepdims=True)

        mp = (iou_max > 0.5).astype(jnp.float32)
        mn = 1.0 - mp
        excl = lax.dot(tri_ref[...], jnp.concatenate([mp, mn], axis=1))
        pos_sel = mp * (pcnt + excl[:, 0:1] < _N_POS).astype(jnp.float32)
        neg_sel = mn * (ncnt + excl[:, 1:2] < _N_NEG).astype(jnp.float32)
        s_ref[_PCNT] = pcnt + jnp.sum(mp)
        s_ref[_NCNT] = ncnt + jnp.sum(mn)

        any_sel = jnp.sum(pos_sel) + jnp.sum(neg_sel) > 0.0

        @pl.when(any_sel)
        def _():
            tcols = lax.broadcasted_iota(jnp.int32, (_RT, _T), 1)
            amax = jnp.min(jnp.where(iou >= iou_max, tcols, _T),
                           axis=1, keepdims=True)
            onehot = (tcols == amax).astype(jnp.float32)

            cls_t = cl_ref[0].astype(jnp.float32)
            cstar = jnp.sum(onehot * cls_t, axis=1, keepdims=True)
            bstar_t = jnp.sum(onehot * bt, axis=1, keepdims=True)
            bstar_l = jnp.sum(onehot * bl, axis=1, keepdims=True)
            bstar_b = jnp.sum(onehot * bbm, axis=1, keepdims=True)
            bstar_r = jnp.sum(onehot * br, axis=1, keepdims=True)

            x = cls_ref[0]
            xmax = jnp.max(x, axis=1, keepdims=True)
            lse = xmax + jnp.log(jnp.sum(jnp.exp(x - xmax), axis=1,
                                         keepdims=True))
            logit0 = x[:, 0:1]
            cstar_i = cstar.astype(jnp.int32)
            ic = lax.broadcasted_iota(jnp.int32, (_RT, _C), 1)
            logit_star = jnp.sum(jnp.where(ic == cstar_i, x, 0.0),
                                 axis=1, keepdims=True)
            pred = jnp.min(jnp.where(x >= xmax, ic, _C),
                           axis=1, keepdims=True)
            ce_pos = lse - logit_star
            ce_neg = lse - logit0
            hit_pos = (pred == cstar_i).astype(jnp.float32)
            hit_neg = (pred == 0).astype(jnp.float32)

            reg = reg_ref[0]
            r01 = jnp.floor(nms[:, 0:2] * _RED) / _RED
            r23 = jnp.ceil(nms[:, 2:4] * _RED) / _RED
            rsum = jnp.zeros((_RT, 1), jnp.float32)
            targets = (bstar_t - r01[:, 0:1], bstar_l - r01[:, 1:2],
                       bstar_b - r23[:, 0:1], bstar_r - r23[:, 1:2])
            for c in range(4):
                d = reg[:, c:c + 1] - targets[c]
                ad = jnp.abs(d)
                rsum = rsum + jnp.where(ad < 1.0, 0.5 * d * d, ad - 0.5)

            s_ref[_CLS] = s_ref[_CLS] + jnp.sum(pos_sel * ce_pos
                                                + neg_sel * ce_neg)
            s_ref[_ACC] = s_ref[_ACC] + jnp.sum(pos_sel * hit_pos
                                                + neg_sel * hit_neg)
            s_ref[_REG] = s_ref[_REG] + jnp.sum(pos_sel * rsum)

            @pl.when(first)
            def _():
                rm = (lax.broadcasted_iota(jnp.int32, (_RT, 1), 0) == 0
                      ).astype(jnp.float32)
                s_ref[_CE0P] = jnp.sum(rm * ce_pos)
                s_ref[_CE0N] = jnp.sum(rm * ce_neg)
                s_ref[_A0P] = jnp.sum(rm * hit_pos)
                s_ref[_A0N] = jnp.sum(rm * hit_neg)
                s_ref[_R0] = jnp.sum(rm * rsum)

    @pl.when(last)
    def _():
        pad_p = _N_POS - jnp.minimum(s_ref[_PCNT], float(_N_POS))
        pad_n = _N_NEG - jnp.minimum(s_ref[_NCNT], float(_N_NEG))
        denom = float(_N_POS + _N_NEG)
        out_cls[...] = jnp.full((1, 1), (s_ref[_CLS] + pad_p * s_ref[_CE0P]
                                         + pad_n * s_ref[_CE0N]) / denom)
        out_acc[...] = jnp.full((1, 1), (s_ref[_ACC] + pad_p * s_ref[_A0P]
                                         + pad_n * s_ref[_A0N]) / denom)
        out_reg[...] = jnp.full((1, 1), (s_ref[_REG] + pad_p * s_ref[_R0])
                                / (4.0 * _N_POS))


def kernel(nms_reg, nms_cls, rcnn_reg, rcnn_cls, bboxes, classes):
    del nms_cls
    bb_t = jnp.transpose(bboxes, (0, 2, 1))
    cl_r = classes.astype(jnp.int32).reshape(_B, 1, _T)
    cls_loss, reg_loss, acc = pl.pallas_call(
        _body,
        grid=(_B, _NR),
        in_specs=[
            pl.BlockSpec((1, _RT, 4), lambda b, i: (b, i, 0)),
            pl.BlockSpec((1, _RT, 4), lambda b, i: (b, i, 0)),
            pl.BlockSpec((1, _RT, _C), lambda b, i: (b, i, 0)),
            pl.BlockSpec((1, 4, _T), lambda b, i: (b, 0, 0)),
            pl.BlockSpec((1, 1, _T), lambda b, i: (b, 0, 0)),
        ],
        out_specs=[pl.BlockSpec((1, 1), lambda b, i: (0, 0))] * 3,
        out_shape=[jax.ShapeDtypeStruct((1, 1), jnp.float32)] * 3,
        scratch_shapes=[pltpu.SMEM((_NSLOT,), jnp.float32),
                        pltpu.VMEM((_RT, _RT), jnp.float32)],
    )(nms_reg, rcnn_reg, rcnn_cls, bb_t, cl_r)
    return (cls_loss.reshape(1), reg_loss.reshape(1), acc.reshape(1))


# speedup vs baseline: 1.0185x; 1.0185x over previous
import jax
import jax.numpy as jnp
from jax import lax
from jax.experimental import pallas as pl
from jax.experimental.pallas import tpu as pltpu

_B, _T, _R, _C = 8, 50, 5000, 81
_RED = 16.0
_N_POS = 128
_N_NEG = 384
_RT = 1000
_NR = _R // _RT

_PCNT, _NCNT, _CLS, _ACC, _REG, _CE0P, _CE0N, _A0P, _A0N, _R0 = range(10)
_NSLOT = 10


def _body(nms_ref, reg_ref, cls_ref, bb_ref, cl_ref,
          out_cls, out_reg, out_acc, s_ref, tri_ref):
    b = pl.program_id(0)
    i = pl.program_id(1)
    first = jnp.logical_and(b == 0, i == 0)
    last = jnp.logical_and(b == _B - 1, i == _NR - 1)

    @pl.when(first)
    def _():
        for k in range(_NSLOT):
            s_ref[k] = 0.0
        ri = lax.broadcasted_iota(jnp.int32, (_RT, _RT), 0)
        ci = lax.broadcasted_iota(jnp.int32, (_RT, _RT), 1)
        tri_ref[...] = (ri > ci).astype(jnp.float32)

    pcnt = s_ref[_PCNT]
    ncnt = s_ref[_NCNT]
    active = jnp.logical_or(pcnt < float(_N_POS), ncnt < float(_N_NEG))

    @pl.when(active)
    def _():
        nms = nms_ref[0]
        top, left = nms[:, 0:1], nms[:, 1:2]
        bot, right = nms[:, 2:3], nms[:, 3:4]
        bb = bb_ref[0]
        bt, bl = bb[0:1, :], bb[1:2, :]
        bbm, br = bb[2:3, :], bb[3:4, :]
        it = jnp.maximum(top, bt)
        il = jnp.maximum(left, bl)
        ib = jnp.minimum(bot, bbm)
        ir = jnp.minimum(right, br)
        inter = jnp.maximum(ib - it, 0.0) * jnp.maximum(ir - il, 0.0)
        area_a = (bot - top) * (right - left)
        area_b = (bbm - bt) * (br - bl)
        iou = inter / (area_a + area_b - inter + 1e-8)
        iou_max = jnp.max(iou, axis=1, keepdims=True)
        tcols = lax.broadcasted_iota(jnp.int32, (_RT, _T), 1)
        amax = jnp.min(jnp.where(iou >= iou_max, tcols, _T),
                       axis=1, keepdims=True)
        onehot = (tcols == amax).astype(jnp.float32)

        cls_t = cl_ref[0].astype(jnp.float32)
        cstar = jnp.sum(onehot * cls_t, axis=1, keepdims=True)
        bstar_t = jnp.sum(onehot * bt, axis=1, keepdims=True)
        bstar_l = jnp.sum(onehot * bl, axis=1, keepdims=True)
        bstar_b = jnp.sum(onehot * bbm, axis=1, keepdims=True)
        bstar_r = jnp.sum(onehot * br, axis=1, keepdims=True)

        x = cls_ref[0]
        xmax = jnp.max(x, axis=1, keepdims=True)
        lse = xmax + jnp.log(jnp.sum(jnp.exp(x - xmax), axis=1,
                                     keepdims=True))
        logit0 = x[:, 0:1]
        cstar_i = cstar.astype(jnp.int32)
        ic = lax.broadcasted_iota(jnp.int32, (_RT, _C), 1)
        logit_star = jnp.sum(jnp.where(ic == cstar_i, x, 0.0),
                             axis=1, keepdims=True)
        pred = jnp.min(jnp.where(x >= xmax, ic, _C), axis=1, keepdims=True)
        ce_pos = lse - logit_star
        ce_neg = lse - logit0
        hit_pos = (pred == cstar_i).astype(jnp.float32)
        hit_neg = (pred == 0).astype(jnp.float32)

        reg = reg_ref[0]
        r01 = jnp.floor(nms[:, 0:2] * _RED) / _RED
        r23 = jnp.ceil(nms[:, 2:4] * _RED) / _RED
        rsum = jnp.zeros((_RT, 1), jnp.float32)
        targets = (bstar_t - r01[:, 0:1], bstar_l - r01[:, 1:2],
                   bstar_b - r23[:, 0:1], bstar_r - r23[:, 1:2])
        for c in range(4):
            d = reg[:, c:c + 1] - targets[c]
            ad = jnp.abs(d)
            rsum = rsum + jnp.where(ad < 1.0, 0.5 * d * d, ad - 0.5)

        mp = (iou_max > 0.5).astype(jnp.float32)
        mn = 1.0 - mp
        excl = lax.dot(tri_ref[...], jnp.concatenate([mp, mn], axis=1))
        pos_sel = mp * (pcnt + excl[:, 0:1] < _N_POS).astype(jnp.float32)
        neg_sel = mn * (ncnt + excl[:, 1:2] < _N_NEG).astype(jnp.float32)

        s_ref[_PCNT] = pcnt + jnp.sum(mp)
        s_ref[_NCNT] = ncnt + jnp.sum(mn)
        s_ref[_CLS] = s_ref[_CLS] + jnp.sum(pos_sel * ce_pos
                                            + neg_sel * ce_neg)
        s_ref[_ACC] = s_ref[_ACC] + jnp.sum(pos_sel * hit_pos
                                            + neg_sel * hit_neg)
        s_ref[_REG] = s_ref[_REG] + jnp.sum(pos_sel * rsum)

        @pl.when(first)
        def _():
            rm = (lax.broadcasted_iota(jnp.int32, (_RT, 1), 0) == 0
                  ).astype(jnp.float32)
            s_ref[_CE0P] = jnp.sum(rm * ce_pos)
            s_ref[_CE0N] = jnp.sum(rm * ce_neg)
            s_ref[_A0P] = jnp.sum(rm * hit_pos)
            s_ref[_A0N] = jnp.sum(rm * hit_neg)
            s_ref[_R0] = jnp.sum(rm * rsum)

    @pl.when(last)
    def _():
        pad_p = _N_POS - jnp.minimum(s_ref[_PCNT], float(_N_POS))
        pad_n = _N_NEG - jnp.minimum(s_ref[_NCNT], float(_N_NEG))
        denom = float(_N_POS + _N_NEG)
        out_cls[...] = jnp.full((1, 1), (s_ref[_CLS] + pad_p * s_ref[_CE0P]
                                         + pad_n * s_ref[_CE0N]) / denom)
        out_acc[...] = jnp.full((1, 1), (s_ref[_ACC] + pad_p * s_ref[_A0P]
                                         + pad_n * s_ref[_A0N]) / denom)
        out_reg[...] = jnp.full((1, 1), (s_ref[_REG] + pad_p * s_ref[_R0])
                                / (4.0 * _N_POS))


def kernel(nms_reg, nms_cls, rcnn_reg, rcnn_cls, bboxes, classes):
    del nms_cls
    bb_t = jnp.transpose(bboxes, (0, 2, 1))
    cl_r = classes.astype(jnp.int32).reshape(_B, 1, _T)
    cls_loss, reg_loss, acc = pl.pallas_call(
        _body,
        grid=(_B, _NR),
        in_specs=[
            pl.BlockSpec((1, _RT, 4), lambda b, i: (b, i, 0)),
            pl.BlockSpec((1, _RT, 4), lambda b, i: (b, i, 0)),
            pl.BlockSpec((1, _RT, _C), lambda b, i: (b, i, 0)),
            pl.BlockSpec((1, 4, _T), lambda b, i: (b, 0, 0)),
            pl.BlockSpec((1, 1, _T), lambda b, i: (b, 0, 0)),
        ],
        out_specs=[pl.BlockSpec((1, 1), lambda b, i: (0, 0))] * 3,
        out_shape=[jax.ShapeDtypeStruct((1, 1), jnp.float32)] * 3,
        scratch_shapes=[pltpu.SMEM((_NSLOT,), jnp.float32),
                        pltpu.VMEM((_RT, _RT), jnp.float32)],
    )(nms_reg, rcnn_reg, rcnn_cls, bb_t, cl_r)
    return (cls_loss.reshape(1), reg_loss.reshape(1), acc.reshape(1))
